# baseline (device time: 26695 ns/iter reference)
import jax
import jax.numpy as jnp
from jax import lax
from jax.experimental import pallas as pl
from jax.experimental.pallas import tpu as pltpu

K = 4


def kernel(x):
    m, n = x.shape
    n_out = n // 2
    h = m // 2
    c = h // K

    def body(x_hbm, out_hbm, xs, stage_send, xl, outbuf,
             xs_sem, xl_sem, loc_sem,
             send_sem1, recv_sem1, send_sem2, recv_sem2):
        my_x = lax.axis_index("x")
        my_y = lax.axis_index("y")
        other_x = 1 - my_x
        other_y = 1 - my_y

        mine_rows = other_y * m + my_x * h
        theirs_rows = other_y * m + other_x * h

        fetch = []
        for k in range(K):
            f = pltpu.make_async_copy(
                x_hbm.at[pl.ds(my_x * h + k * c, c),
                         pl.ds(other_y * n_out, n_out)],
                xs.at[k], xs_sem.at[k],
            )
            f.start()
            fetch.append(f)
        locf = pltpu.make_async_copy(
            x_hbm.at[:, pl.ds(my_y * n_out, n_out)], xl, xl_sem,
        )
        locf.start()

        barrier = pltpu.get_barrier_semaphore()
        pl.semaphore_signal(barrier, inc=1, device_id=(my_x, other_y),
                            device_id_type=pl.DeviceIdType.MESH)
        pl.semaphore_signal(barrier, inc=1, device_id=(other_x, my_y),
                            device_id_type=pl.DeviceIdType.MESH)
        pl.semaphore_wait(barrier, 2)

        send1, recv1 = [], []
        for k in range(K):
            fetch[k].wait()
            stage_send[k] = xs[k].astype(jnp.bfloat16)
            s = pltpu.make_async_remote_copy(
                src_ref=stage_send.at[k],
                dst_ref=out_hbm.at[pl.ds(my_y * m + my_x * h + k * c, c), :],
                send_sem=send_sem1.at[k], recv_sem=recv_sem1.at[k],
                device_id=(my_x, other_y),
                device_id_type=pl.DeviceIdType.MESH,
            )
            s.start()
            send1.append(s)
            recv1.append(pltpu.make_async_remote_copy(
                src_ref=stage_send.at[k],
                dst_ref=out_hbm.at[pl.ds(mine_rows + k * c, c), :],
                send_sem=send_sem1.at[k], recv_sem=recv_sem1.at[k],
                device_id=(my_x, other_y),
                device_id_type=pl.DeviceIdType.MESH,
            ))

        locf.wait()
        outbuf[...] = xl[...].astype(jnp.bfloat16)
        loc = pltpu.make_async_copy(
            outbuf, out_hbm.at[pl.ds(my_y * m, m), :], loc_sem,
        )
        loc.start()

        send2, recv2 = [], []
        for k in range(K):
            recv1[k].wait_recv()
            s = pltpu.make_async_remote_copy(
                src_ref=out_hbm.at[pl.ds(mine_rows + k * c, c), :],
                dst_ref=out_hbm.at[pl.ds(mine_rows + k * c, c), :],
                send_sem=send_sem2.at[k], recv_sem=recv_sem2.at[k],
                device_id=(other_x, my_y),
                device_id_type=pl.DeviceIdType.MESH,
            )
            s.start()
            send2.append(s)
            recv2.append(pltpu.make_async_remote_copy(
                src_ref=stage_send.at[k],
                dst_ref=out_hbm.at[pl.ds(theirs_rows + k * c, c), :],
                send_sem=send_sem2.at[k], recv_sem=recv_sem2.at[k],
                device_id=(other_x, my_y),
                device_id_type=pl.DeviceIdType.MESH,
            ))

        for k in range(K):
            recv2[k].wait_recv()
        loc.wait()
        for k in range(K):
            send1[k].wait_send()
            send2[k].wait_send()

    return pl.pallas_call(
        body,
        out_shape=jax.ShapeDtypeStruct((2 * m, n_out), jnp.bfloat16),
        in_specs=[pl.BlockSpec(memory_space=pl.ANY)],
        out_specs=pl.BlockSpec(memory_space=pl.ANY),
        scratch_shapes=[
            pltpu.VMEM((K, c, n_out), jnp.float32),
            pltpu.VMEM((K, c, n_out), jnp.bfloat16),
            pltpu.VMEM((m, n_out), jnp.float32),
            pltpu.VMEM((m, n_out), jnp.bfloat16),
            pltpu.SemaphoreType.DMA((K,)),
            pltpu.SemaphoreType.DMA,
            pltpu.SemaphoreType.DMA,
            pltpu.SemaphoreType.DMA((K,)),
            pltpu.SemaphoreType.DMA((K,)),
            pltpu.SemaphoreType.DMA((K,)),
            pltpu.SemaphoreType.DMA((K,)),
        ],
        compiler_params=pltpu.CompilerParams(collective_id=0),
    )(x)


# device time: 24356 ns/iter; 1.0960x vs baseline; 1.0960x over previous
import jax
import jax.numpy as jnp
from jax import lax
from jax.experimental import pallas as pl
from jax.experimental.pallas import tpu as pltpu

K = 8


def kernel(x):
    m, n = x.shape
    n_out = n // 2
    h = m // 2
    c = h // K

    def body(x_ref, out_ref, stage_send,
             send_sem1, recv_sem1, send_sem2, recv_sem2):
        my_x = lax.axis_index("x")
        my_y = lax.axis_index("y")
        other_x = 1 - my_x
        other_y = 1 - my_y

        mine_rows = other_y * m + my_x * h
        theirs_rows = other_y * m + other_x * h

        barrier = pltpu.get_barrier_semaphore()
        pl.semaphore_signal(barrier, inc=1, device_id=(my_x, other_y),
                            device_id_type=pl.DeviceIdType.MESH)
        pl.semaphore_signal(barrier, inc=1, device_id=(other_x, my_y),
                            device_id_type=pl.DeviceIdType.MESH)
        pl.semaphore_wait(barrier, 2)

        send1, recv1 = [], []
        for k in range(K):
            stage_send[k] = x_ref[
                pl.ds(my_x * h + k * c, c), pl.ds(other_y * n_out, n_out)
            ].astype(jnp.bfloat16)
            s = pltpu.make_async_remote_copy(
                src_ref=stage_send.at[k],
                dst_ref=out_ref.at[pl.ds(my_y * m + my_x * h + k * c, c), :],
                send_sem=send_sem1.at[k], recv_sem=recv_sem1.at[k],
                device_id=(my_x, other_y),
                device_id_type=pl.DeviceIdType.MESH,
            )
            s.start()
            send1.append(s)
            recv1.append(pltpu.make_async_remote_copy(
                src_ref=stage_send.at[k],
                dst_ref=out_ref.at[pl.ds(mine_rows + k * c, c), :],
                send_sem=send_sem1.at[k], recv_sem=recv_sem1.at[k],
                device_id=(my_x, other_y),
                device_id_type=pl.DeviceIdType.MESH,
            ))

        out_ref[pl.ds(my_y * m, m), :] = x_ref[
            :, pl.ds(my_y * n_out, n_out)
        ].astype(jnp.bfloat16)

        send2, recv2 = [], []
        for k in range(K):
            recv1[k].wait_recv()
            s = pltpu.make_async_remote_copy(
                src_ref=out_ref.at[pl.ds(mine_rows + k * c, c), :],
                dst_ref=out_ref.at[pl.ds(mine_rows + k * c, c), :],
                send_sem=send_sem2.at[k], recv_sem=recv_sem2.at[k],
                device_id=(other_x, my_y),
                device_id_type=pl.DeviceIdType.MESH,
            )
            s.start()
            send2.append(s)
            recv2.append(pltpu.make_async_remote_copy(
                src_ref=stage_send.at[k],
                dst_ref=out_ref.at[pl.ds(theirs_rows + k * c, c), :],
                send_sem=send_sem2.at[k], recv_sem=recv_sem2.at[k],
                device_id=(other_x, my_y),
                device_id_type=pl.DeviceIdType.MESH,
            ))

        for k in range(K):
            recv2[k].wait_recv()
        for k in range(K):
            send1[k].wait_send()
            send2[k].wait_send()

    return pl.pallas_call(
        body,
        out_shape=jax.ShapeDtypeStruct((2 * m, n_out), jnp.bfloat16),
        in_specs=[pl.BlockSpec(memory_space=pltpu.VMEM)],
        out_specs=pl.BlockSpec(memory_space=pltpu.VMEM),
        scratch_shapes=[
            pltpu.VMEM((K, c, n_out), jnp.bfloat16),
            pltpu.SemaphoreType.DMA((K,)),
            pltpu.SemaphoreType.DMA((K,)),
            pltpu.SemaphoreType.DMA((K,)),
            pltpu.SemaphoreType.DMA((K,)),
        ],
        compiler_params=pltpu.CompilerParams(collective_id=0),
    )(x)


# device time: 23954 ns/iter; 1.1144x vs baseline; 1.0168x over previous
import jax
import jax.numpy as jnp
from jax import lax
from jax.experimental import pallas as pl
from jax.experimental.pallas import tpu as pltpu

K = 16


def kernel(x):
    m, n = x.shape
    n_out = n // 2
    h = m // 2
    c = h // K

    def body(x_ref, out_ref, stage_send,
             send_sem1, recv_sem1, send_sem2, recv_sem2):
        my_x = lax.axis_index("x")
        my_y = lax.axis_index("y")
        other_x = 1 - my_x
        other_y = 1 - my_y

        mine_rows = other_y * m + my_x * h
        theirs_rows = other_y * m + other_x * h

        barrier = pltpu.get_barrier_semaphore()
        pl.semaphore_signal(barrier, inc=1, device_id=(my_x, other_y),
                            device_id_type=pl.DeviceIdType.MESH)
        pl.semaphore_signal(barrier, inc=1, device_id=(other_x, my_y),
                            device_id_type=pl.DeviceIdType.MESH)
        pl.semaphore_wait(barrier, 2)

        send1, recv1 = [], []
        for k in range(K):
            stage_send[k] = x_ref[
                pl.ds(my_x * h + k * c, c), pl.ds(other_y * n_out, n_out)
            ].astype(jnp.bfloat16)
            s = pltpu.make_async_remote_copy(
                src_ref=stage_send.at[k],
                dst_ref=out_ref.at[pl.ds(my_y * m + my_x * h + k * c, c), :],
                send_sem=send_sem1.at[k], recv_sem=recv_sem1.at[k],
                device_id=(my_x, other_y),
                device_id_type=pl.DeviceIdType.MESH,
            )
            s.start()
            send1.append(s)
            recv1.append(pltpu.make_async_remote_copy(
                src_ref=stage_send.at[k],
                dst_ref=out_ref.at[pl.ds(mine_rows + k * c, c), :],
                send_sem=send_sem1.at[k], recv_sem=recv_sem1.at[k],
                device_id=(my_x, other_y),
                device_id_type=pl.DeviceIdType.MESH,
            ))

        out_ref[pl.ds(my_y * m, m), :] = x_ref[
            :, pl.ds(my_y * n_out, n_out)
        ].astype(jnp.bfloat16)

        send2, recv2 = [], []
        for k in range(K):
            recv1[k].wait_recv()
            s = pltpu.make_async_remote_copy(
                src_ref=out_ref.at[pl.ds(mine_rows + k * c, c), :],
                dst_ref=out_ref.at[pl.ds(mine_rows + k * c, c), :],
                send_sem=send_sem2.at[k], recv_sem=recv_sem2.at[k],
                device_id=(other_x, my_y),
                device_id_type=pl.DeviceIdType.MESH,
            )
            s.start()
            send2.append(s)
            recv2.append(pltpu.make_async_remote_copy(
                src_ref=stage_send.at[k],
                dst_ref=out_ref.at[pl.ds(theirs_rows + k * c, c), :],
                send_sem=send_sem2.at[k], recv_sem=recv_sem2.at[k],
                device_id=(other_x, my_y),
                device_id_type=pl.DeviceIdType.MESH,
            ))

        for k in range(K):
            recv2[k].wait_recv()
        for k in range(K):
            send1[k].wait_send()
            send2[k].wait_send()

    return pl.pallas_call(
        body,
        out_shape=jax.ShapeDtypeStruct((2 * m, n_out), jnp.bfloat16),
        in_specs=[pl.BlockSpec(memory_space=pltpu.VMEM)],
        out_specs=pl.BlockSpec(memory_space=pltpu.VMEM),
        scratch_shapes=[
            pltpu.VMEM((K, c, n_out), jnp.bfloat16),
            pltpu.SemaphoreType.DMA((K,)),
            pltpu.SemaphoreType.DMA((K,)),
            pltpu.SemaphoreType.DMA((K,)),
            pltpu.SemaphoreType.DMA((K,)),
        ],
        compiler_params=pltpu.CompilerParams(collective_id=0),
    )(x)
